# baseline (device time: 11296 ns/iter reference)
import jax
import jax.numpy as jnp
from jax import lax
from jax.experimental import pallas as pl
from jax.experimental.pallas import tpu as pltpu

N_DEV = 4


def kernel(x, dy, gamma):
    m, d = x.shape

    def body(x_ref, dy_ref, out_ref, comm_ref, send_sems, recv_sems):
        my = lax.axis_index("i")

        xv = x_ref[:, :]
        dyv = dy_ref[:, :]
        mu = jnp.mean(xv, axis=1, keepdims=True)
        var = jnp.mean((xv - mu) * (xv - mu), axis=1, keepdims=True)
        rstd = lax.rsqrt(var + 1e-5)
        xhat = (xv - mu) * rstd
        dgamma = jnp.sum(dyv * xhat, axis=0)
        dbeta = jnp.sum(dyv, axis=0)
        comm_ref[0, 0, :] = dgamma
        comm_ref[0, 1, :] = dbeta

        barrier_sem = pltpu.get_barrier_semaphore()
        for k in range(1, N_DEV):
            pl.semaphore_signal(
                barrier_sem, inc=1,
                device_id=((my + k) % N_DEV,),
                device_id_type=pl.DeviceIdType.MESH,
            )
        pl.semaphore_wait(barrier_sem, N_DEV - 1)

        sends = []
        for k in range(1, N_DEV):
            rdma = pltpu.make_async_remote_copy(
                src_ref=comm_ref.at[0],
                dst_ref=comm_ref.at[k],
                send_sem=send_sems.at[k - 1],
                recv_sem=recv_sems.at[k - 1],
                device_id=((my + k) % N_DEV,),
                device_id_type=pl.DeviceIdType.MESH,
            )
            rdma.start()
            sends.append(rdma)

        for rdma in sends:
            rdma.wait_recv()

        out_ref[:, :] = (
            comm_ref[0] + comm_ref[1] + comm_ref[2] + comm_ref[3]
        )

        for rdma in sends:
            rdma.wait_send()

    return pl.pallas_call(
        body,
        out_shape=jax.ShapeDtypeStruct((2, d), jnp.float32),
        in_specs=[
            pl.BlockSpec(memory_space=pltpu.VMEM),
            pl.BlockSpec(memory_space=pltpu.VMEM),
        ],
        out_specs=pl.BlockSpec(memory_space=pltpu.VMEM),
        scratch_shapes=[
            pltpu.VMEM((N_DEV, 2, d), jnp.float32),
            pltpu.SemaphoreType.DMA((N_DEV - 1,)),
            pltpu.SemaphoreType.DMA((N_DEV - 1,)),
        ],
        compiler_params=pltpu.CompilerParams(collective_id=0),
    )(x, dy)
